# Initial kernel scaffold; baseline (speedup 1.0000x reference)
import jax
import jax.numpy as jnp
from jax.experimental import pallas as pl


def kernel(x, edge_index, batch, W_conv, b_conv, W_lin, b_lin):
    raise NotImplementedError("write your pallas kernel here")


# speedup vs baseline: 25.2368x; 25.2368x over previous
import functools

import jax
import jax.numpy as jnp
from jax import lax
from jax.experimental import pallas as pl
from jax.experimental.pallas import tpu as pltpu
from jax.experimental.pallas import tpu_sc as plsc

N = 10000
E = 320000
DIN = 128
DH = 64
DOUT = 32
G = 64

NC = 2
NS = 16
L = 16
NW = NC * NS

CHUNK = 128
K = 80
EPAD = NW * K * CHUNK
NPAD = 10240
RPT = NPAD // NS
TRASH = NPAD - 1
NPT = N // NS

RB = 400
NB = N // RB

def _fill_rows(ref, nrows, vec):
  ncols = ref.shape[1] // L

  def body(i, _):
    for j in range(ncols):
      ref[i, pl.ds(j * L, L)] = vec
    return 0

  lax.fori_loop(0, nrows, body, 0)


def _deg_body(col_hbm, deg_hbm, col_v, ones_v, zero_v, deg_sh):
  cid = lax.axis_index("c")
  sid = lax.axis_index("s")
  wid = cid * NS + sid

  lane = lax.iota(jnp.int32, L)
  one0 = jnp.where(lane == 0, 1.0, 0.0).astype(jnp.float32)
  z16 = jnp.zeros((L,), jnp.float32)
  _fill_rows(ones_v, CHUNK, one0)
  _fill_rows(zero_v, CHUNK, z16)

  for j in range(RPT // CHUNK):
    pltpu.sync_copy(zero_v, deg_sh.at[pl.ds(sid * RPT + j * CHUNK, CHUNK)])
  plsc.subcore_barrier()

  pltpu.sync_copy(col_hbm.at[wid], col_v)

  def step(k, _):
    pltpu.sync_copy(ones_v, deg_sh.at[col_v.at[k]], add=True)
    return 0

  lax.fori_loop(0, K, step, 0)
  plsc.subcore_barrier()

  pltpu.sync_copy(deg_sh.at[pl.ds(sid * RPT, RPT)],
                  deg_hbm.at[cid, pl.ds(sid * RPT, RPT)])


def _msg_body(g_hbm, row_hbm, col_hbm, s_hbm,
              row_v, col_v, buf0, buf1, zero_v, s_sh, sem0, sem1):
  cid = lax.axis_index("c")
  sid = lax.axis_index("s")
  wid = cid * NS + sid

  z16 = jnp.zeros((L,), jnp.float32)
  _fill_rows(zero_v, CHUNK, z16)
  for j in range(RPT // CHUNK):
    pltpu.sync_copy(zero_v, s_sh.at[pl.ds(sid * RPT + j * CHUNK, CHUNK)])
  plsc.subcore_barrier()

  pltpu.sync_copy(row_hbm.at[wid], row_v)
  pltpu.sync_copy(col_hbm.at[wid], col_v)

  pltpu.async_copy(g_hbm.at[row_v.at[0]], buf0, sem0)

  def step(t, _):
    kk = 2 * t
    pltpu.async_copy(g_hbm.at[row_v.at[kk + 1]], buf1, sem1)
    pltpu.make_async_copy(g_hbm.at[row_v.at[kk]], buf0, sem0).wait()
    pltpu.sync_copy(buf0, s_sh.at[col_v.at[kk]], add=True)

    @pl.when(kk + 2 < K)
    def _():
      pltpu.async_copy(g_hbm.at[row_v.at[kk + 2]], buf0, sem0)

    pltpu.make_async_copy(g_hbm.at[row_v.at[kk + 1]], buf1, sem1).wait()
    pltpu.sync_copy(buf1, s_sh.at[col_v.at[kk + 1]], add=True)
    return 0

  lax.fori_loop(0, K // 2, step, 0)
  plsc.subcore_barrier()

  pltpu.sync_copy(s_sh.at[pl.ds(sid * RPT, RPT)],
                  s_hbm.at[cid, pl.ds(sid * RPT, RPT)])


def _tca_body(x_ref, w_ref, deg_ref, g_ref, dinv_ref):
  d = deg_ref[0, :, 0:1] + deg_ref[1, :, 0:1] + 1.0
  di = lax.rsqrt(d)
  h = jnp.dot(x_ref[...], w_ref[...], preferred_element_type=jnp.float32)
  g_ref[...] = h * di
  dinv_ref[...] = di


def _tcb_body(s_ref, g_ref, dinv_ref, batch_ref, bconv_ref, wlin_ref,
              blin_ref, out_ref, sums, cnts):
  i = pl.program_id(0)

  @pl.when(i == 0)
  def _():
    sums[...] = jnp.zeros_like(sums)
    cnts[...] = jnp.zeros_like(cnts)

  s = s_ref[0] + s_ref[1]
  o = dinv_ref[...] * (s + g_ref[...]) + bconv_ref[...]
  o = jnp.where(o >= 0, o, 0.01 * o)

  b = batch_ref[0]
  gid = lax.broadcasted_iota(jnp.int32, (G, RB), 0)
  pt = jnp.where(gid == b, 1.0, 0.0).astype(jnp.float32)
  sums[...] += jnp.dot(pt, o, preferred_element_type=jnp.float32)
  cnts[...] += jnp.sum(pt, axis=1, keepdims=True)

  @pl.when(i == NB - 1)
  def _():
    pooled = sums[...] / jnp.maximum(cnts[...], 1.0)
    emb = jnp.dot(pooled, wlin_ref[...], preferred_element_type=jnp.float32)
    out_ref[...] = jnp.tanh(emb + blin_ref[...])


_tca_call = pl.pallas_call(
    _tca_body,
    grid=(NB,),
    in_specs=[
        pl.BlockSpec((RB, DIN), lambda i: (i, 0)),
        pl.BlockSpec((DIN, DH), lambda i: (0, 0)),
        pl.BlockSpec((NC, RB, L), lambda i: (0, i, 0)),
    ],
    out_specs=[
        pl.BlockSpec((RB, DH), lambda i: (i, 0)),
        pl.BlockSpec((RB, 1), lambda i: (i, 0)),
    ],
    out_shape=[
        jax.ShapeDtypeStruct((N, DH), jnp.float32),
        jax.ShapeDtypeStruct((N, 1), jnp.float32),
    ],
)

_tcb_call = pl.pallas_call(
    _tcb_body,
    grid=(NB,),
    in_specs=[
        pl.BlockSpec((NC, RB, DH), lambda i: (0, i, 0)),
        pl.BlockSpec((RB, DH), lambda i: (i, 0)),
        pl.BlockSpec((RB, 1), lambda i: (i, 0)),
        pl.BlockSpec((1, 1, RB), lambda i: (i, 0, 0)),
        pl.BlockSpec((1, DH), lambda i: (0, 0)),
        pl.BlockSpec((DH, DOUT), lambda i: (0, 0)),
        pl.BlockSpec((1, DOUT), lambda i: (0, 0)),
    ],
    out_specs=pl.BlockSpec((G, DOUT), lambda i: (0, 0)),
    out_shape=jax.ShapeDtypeStruct((G, DOUT), jnp.float32),
    scratch_shapes=[
        pltpu.VMEM((G, G), jnp.float32),
        pltpu.VMEM((G, 1), jnp.float32),
    ],
)


@functools.lru_cache(maxsize=1)
def _sc_kernels():
  mesh = plsc.VectorSubcoreMesh(
      core_axis_name="c", subcore_axis_name="s", num_cores=NC, num_subcores=NS)
  params = pltpu.CompilerParams(use_tc_tiling_on_sc=False)
  deg_kernel = pl.kernel(
      _deg_body,
      out_type=jax.ShapeDtypeStruct((NC, NPAD, L), jnp.float32),
      mesh=mesh,
      compiler_params=params,
      scratch_types=[
          pltpu.VMEM((K, CHUNK), jnp.int32),
          pltpu.VMEM((CHUNK, L), jnp.float32),
          pltpu.VMEM((CHUNK, L), jnp.float32),
          pltpu.VMEM_SHARED((NPAD, L), jnp.float32),
      ],
  )
  msg_kernel = pl.kernel(
      _msg_body,
      out_type=jax.ShapeDtypeStruct((NC, NPAD, DH), jnp.float32),
      mesh=mesh,
      compiler_params=params,
      scratch_types=[
          pltpu.VMEM((K, CHUNK), jnp.int32),
          pltpu.VMEM((K, CHUNK), jnp.int32),
          pltpu.VMEM((CHUNK, DH), jnp.float32),
          pltpu.VMEM((CHUNK, DH), jnp.float32),
          pltpu.VMEM((CHUNK, DH), jnp.float32),
          pltpu.VMEM_SHARED((NPAD, DH), jnp.float32),
          pltpu.SemaphoreType.DMA,
          pltpu.SemaphoreType.DMA,
      ],
  )
  return deg_kernel, msg_kernel


def kernel(x, edge_index, batch, W_conv, b_conv, W_lin, b_lin):
  deg_kernel, msg_kernel = _sc_kernels()
  row = edge_index[0]
  col = edge_index[1]
  rowp = jnp.concatenate(
      [row, jnp.zeros((EPAD - E,), jnp.int32)]).reshape(NW, K, CHUNK)
  colp = jnp.concatenate(
      [col, jnp.full((EPAD - E,), TRASH, jnp.int32)]).reshape(NW, K, CHUNK)

  deg = deg_kernel(colp)
  g, dinv = _tca_call(x, W_conv, deg)
  s = msg_kernel(g, rowp, colp)
  emb = _tcb_call(s, g, dinv, batch.reshape(NB, 1, RB),
                  b_conv.reshape(1, DH), W_lin, b_lin.reshape(1, DOUT))
  return emb


# speedup vs baseline: 49.2828x; 1.9528x over previous
import functools

import jax
import jax.numpy as jnp
from jax import lax
from jax.experimental import pallas as pl
from jax.experimental.pallas import tpu as pltpu
from jax.experimental.pallas import tpu_sc as plsc

N = 10000
E = 320000
DIN = 128
DH = 64
DOUT = 32
G = 64

NC = 2
NS = 16
L = 16
NW = NC * NS

CHUNK = 128
NCHUNKS = E // CHUNK
KMAX = 80
KLAST = NCHUNKS - (NW - 1) * KMAX
NPAD = 10240
RPT = NPAD // NS

RB = 400
NB = N // RB

def _fill_rows(ref, nrows, vec):
  ncols = ref.shape[1] // L

  def body(i, _):
    for j in range(ncols):
      ref[i, pl.ds(j * L, L)] = vec
    return 0

  lax.fori_loop(0, nrows, body, 0)


def _load_edge_chunks(e3_hbm, which, wid, dst_v):
  @pl.when(wid < NW - 1)
  def _():
    pltpu.sync_copy(e3_hbm.at[which, pl.ds(wid * KMAX, KMAX)], dst_v)

  @pl.when(wid == NW - 1)
  def _():
    pltpu.sync_copy(e3_hbm.at[which, pl.ds((NW - 1) * KMAX, KLAST)],
                    dst_v.at[pl.ds(0, KLAST)])


def _deg_body(e3_hbm, deg_hbm, col_v, ones_v, zero_v, deg_sh):
  cid = lax.axis_index("c")
  sid = lax.axis_index("s")
  wid = cid * NS + sid
  nk = jnp.where(wid == NW - 1, KLAST, KMAX)

  lane = lax.iota(jnp.int32, L)
  one0 = jnp.where(lane == 0, 1.0, 0.0).astype(jnp.float32)
  z16 = jnp.zeros((L,), jnp.float32)
  _fill_rows(ones_v, CHUNK, one0)
  _fill_rows(zero_v, CHUNK, z16)

  for j in range(RPT // CHUNK):
    pltpu.sync_copy(zero_v, deg_sh.at[pl.ds(sid * RPT + j * CHUNK, CHUNK)])
  plsc.subcore_barrier()

  _load_edge_chunks(e3_hbm, 1, wid, col_v)

  def step(k, _):
    pltpu.sync_copy(ones_v, deg_sh.at[col_v.at[k]], add=True)
    return 0

  lax.fori_loop(0, nk, step, 0)
  plsc.subcore_barrier()

  pltpu.sync_copy(deg_sh.at[pl.ds(sid * RPT, RPT)],
                  deg_hbm.at[cid, pl.ds(sid * RPT, RPT)])


def _msg_body(g_hbm, e3_hbm, s_hbm,
              row_v, col_v, buf0, buf1, zero_v, s_sh, sem0, sem1):
  cid = lax.axis_index("c")
  sid = lax.axis_index("s")
  wid = cid * NS + sid
  nk = jnp.where(wid == NW - 1, KLAST, KMAX)

  z16 = jnp.zeros((L,), jnp.float32)
  _fill_rows(zero_v, CHUNK, z16)
  for j in range(RPT // CHUNK):
    pltpu.sync_copy(zero_v, s_sh.at[pl.ds(sid * RPT + j * CHUNK, CHUNK)])
  plsc.subcore_barrier()

  _load_edge_chunks(e3_hbm, 0, wid, row_v)
  _load_edge_chunks(e3_hbm, 1, wid, col_v)

  pltpu.async_copy(g_hbm.at[row_v.at[0]], buf0, sem0)

  def step(t, _):
    kk = 2 * t
    pltpu.async_copy(g_hbm.at[row_v.at[kk + 1]], buf1, sem1)
    pltpu.make_async_copy(g_hbm.at[row_v.at[kk]], buf0, sem0).wait()
    pltpu.sync_copy(buf0, s_sh.at[col_v.at[kk]], add=True)

    @pl.when(kk + 2 < nk)
    def _():
      pltpu.async_copy(g_hbm.at[row_v.at[kk + 2]], buf0, sem0)

    pltpu.make_async_copy(g_hbm.at[row_v.at[kk + 1]], buf1, sem1).wait()
    pltpu.sync_copy(buf1, s_sh.at[col_v.at[kk + 1]], add=True)
    return 0

  lax.fori_loop(0, nk // 2, step, 0)
  plsc.subcore_barrier()

  pltpu.sync_copy(s_sh.at[pl.ds(sid * RPT, RPT)],
                  s_hbm.at[pl.ds(sid * RPT, RPT), pl.ds(cid * DH, DH)])


def _tca0_body(x_ref, w_ref, h_ref):
  h_ref[...] = jnp.dot(x_ref[...], w_ref[...],
                       preferred_element_type=jnp.float32)


def _dinv_col(deg_ref):
  d = deg_ref[0, :, 0:1] + deg_ref[1, :, 0:1] + 1.0
  return lax.rsqrt(d)


def _tca1_body(h_ref, deg_ref, g_ref):
  g_ref[...] = h_ref[...] * _dinv_col(deg_ref)


def _tcb_body(s_ref, g_ref, deg_ref, batch_ref, bconv_ref, wlin_ref,
              blin_ref, out_ref, sums, cnts):
  i = pl.program_id(0)

  @pl.when(i == 0)
  def _():
    sums[...] = jnp.zeros_like(sums)
    cnts[...] = jnp.zeros_like(cnts)

  s = s_ref[:, 0:DH] + s_ref[:, DH:2 * DH]
  o = _dinv_col(deg_ref) * (s + g_ref[...]) + bconv_ref[...]
  o = jnp.where(o >= 0, o, 0.01 * o)

  b = batch_ref[0]
  gid = lax.broadcasted_iota(jnp.int32, (G, RB), 0)
  pt = jnp.where(gid == b, 1.0, 0.0).astype(jnp.float32)
  sums[...] += jnp.dot(pt, o, preferred_element_type=jnp.float32)
  cnts[...] += jnp.sum(pt, axis=1, keepdims=True)

  @pl.when(i == NB - 1)
  def _():
    pooled = sums[...] / jnp.maximum(cnts[...], 1.0)
    emb = jnp.dot(pooled, wlin_ref[...], preferred_element_type=jnp.float32)
    out_ref[...] = jnp.tanh(emb + blin_ref[...])


_tca0_call = pl.pallas_call(
    _tca0_body,
    grid=(NB,),
    in_specs=[
        pl.BlockSpec((RB, DIN), lambda i: (i, 0)),
        pl.BlockSpec((DIN, DH), lambda i: (0, 0)),
    ],
    out_specs=pl.BlockSpec((RB, DH), lambda i: (i, 0)),
    out_shape=jax.ShapeDtypeStruct((N, DH), jnp.float32),
)

_tca1_call = pl.pallas_call(
    _tca1_body,
    grid=(NB,),
    in_specs=[
        pl.BlockSpec((RB, DH), lambda i: (i, 0)),
        pl.BlockSpec((NC, RB, L), lambda i: (0, i, 0)),
    ],
    out_specs=pl.BlockSpec((RB, DH), lambda i: (i, 0)),
    out_shape=jax.ShapeDtypeStruct((N, DH), jnp.float32),
)

_tcb_call = pl.pallas_call(
    _tcb_body,
    grid=(NB,),
    in_specs=[
        pl.BlockSpec((RB, NC * DH), lambda i: (i, 0)),
        pl.BlockSpec((RB, DH), lambda i: (i, 0)),
        pl.BlockSpec((NC, RB, L), lambda i: (0, i, 0)),
        pl.BlockSpec((1, 1, RB), lambda i: (i, 0, 0)),
        pl.BlockSpec((1, DH), lambda i: (0, 0)),
        pl.BlockSpec((DH, DOUT), lambda i: (0, 0)),
        pl.BlockSpec((1, DOUT), lambda i: (0, 0)),
    ],
    out_specs=pl.BlockSpec((G, DOUT), lambda i: (0, 0)),
    out_shape=jax.ShapeDtypeStruct((G, DOUT), jnp.float32),
    scratch_shapes=[
        pltpu.VMEM((G, G), jnp.float32),
        pltpu.VMEM((G, 1), jnp.float32),
    ],
)


@functools.lru_cache(maxsize=1)
def _sc_kernels():
  mesh = plsc.VectorSubcoreMesh(
      core_axis_name="c", subcore_axis_name="s", num_cores=NC, num_subcores=NS)
  params = pltpu.CompilerParams(use_tc_tiling_on_sc=False)
  deg_kernel = pl.kernel(
      _deg_body,
      out_type=jax.ShapeDtypeStruct((NC, NPAD, L), jnp.float32),
      mesh=mesh,
      compiler_params=params,
      scratch_types=[
          pltpu.VMEM((KMAX, CHUNK), jnp.int32),
          pltpu.VMEM((CHUNK, L), jnp.float32),
          pltpu.VMEM((CHUNK, L), jnp.float32),
          pltpu.VMEM_SHARED((NPAD, L), jnp.float32),
      ],
  )
  msg_kernel = pl.kernel(
      _msg_body,
      out_type=jax.ShapeDtypeStruct((NPAD, NC * DH), jnp.float32),
      mesh=mesh,
      compiler_params=params,
      scratch_types=[
          pltpu.VMEM((KMAX, CHUNK), jnp.int32),
          pltpu.VMEM((KMAX, CHUNK), jnp.int32),
          pltpu.VMEM((CHUNK, DH), jnp.float32),
          pltpu.VMEM((CHUNK, DH), jnp.float32),
          pltpu.VMEM((CHUNK, DH), jnp.float32),
          pltpu.VMEM_SHARED((NPAD, DH), jnp.float32),
          pltpu.SemaphoreType.DMA,
          pltpu.SemaphoreType.DMA,
      ],
  )
  return deg_kernel, msg_kernel


def kernel(x, edge_index, batch, W_conv, b_conv, W_lin, b_lin):
  deg_kernel, msg_kernel = _sc_kernels()
  e3 = edge_index.reshape(2, NCHUNKS, CHUNK)

  h = _tca0_call(x, W_conv)
  deg = deg_kernel(e3)
  g = _tca1_call(h, deg)
  s = msg_kernel(g, e3)
  emb = _tcb_call(s, g, deg, batch.reshape(NB, 1, RB),
                  b_conv.reshape(1, DH), W_lin, b_lin.reshape(1, DOUT))
  return emb


# speedup vs baseline: 63.5350x; 1.2892x over previous
import functools

import jax
import jax.numpy as jnp
from jax import lax
from jax.experimental import pallas as pl
from jax.experimental.pallas import tpu as pltpu
from jax.experimental.pallas import tpu_sc as plsc

N = 10000
E = 320000
DIN = 128
DH = 64
DOUT = 32
G = 64

NC = 2
NS = 16
L = 16
NW = NC * NS

CHUNK = 128
NCHUNKS = E // CHUNK
KMAX = 80
KLAST = NCHUNKS - (NW - 1) * KMAX
NPAD = 10240
RPT = NPAD // NS

RB0 = 400
NB0 = N // RB0
RB = 1024
NB = NPAD // RB
RBS = RB // 128

def _fill_rows(ref, nrows, vec):
  ncols = ref.shape[1] // L

  def body(i, _):
    for j in range(ncols):
      ref[i, pl.ds(j * L, L)] = vec
    return 0

  lax.fori_loop(0, nrows, body, 0)


def _load_edge_chunks(e3_hbm, which, wid, dst_v):
  @pl.when(wid < NW - 1)
  def _():
    pltpu.sync_copy(e3_hbm.at[which, pl.ds(wid * KMAX, KMAX)], dst_v)

  @pl.when(wid == NW - 1)
  def _():
    pltpu.sync_copy(e3_hbm.at[which, pl.ds((NW - 1) * KMAX, KLAST)],
                    dst_v.at[pl.ds(0, KLAST)])


def _deg_body(e3_hbm, deg_hbm, col_v, ones_v, zero_v, deg_sh):
  cid = lax.axis_index("c")
  sid = lax.axis_index("s")
  wid = cid * NS + sid
  nk = jnp.where(wid == NW - 1, KLAST, KMAX)

  one16 = jnp.full((L,), 1.0, jnp.float32)
  z16 = jnp.zeros((L,), jnp.float32)
  for j in range(CHUNK // L):
    ones_v[pl.ds(j * L, L)] = one16
  for j in range(RPT // L):
    zero_v[pl.ds(j * L, L)] = z16

  pltpu.sync_copy(zero_v, deg_sh.at[pl.ds(sid * RPT, RPT)])
  plsc.subcore_barrier()

  _load_edge_chunks(e3_hbm, 1, wid, col_v)

  def step(k, _):
    pltpu.sync_copy(ones_v, deg_sh.at[col_v.at[k]], add=True)
    return 0

  lax.fori_loop(0, nk, step, 0)
  plsc.subcore_barrier()

  pltpu.sync_copy(deg_sh.at[pl.ds(sid * RPT, RPT)],
                  deg_hbm.at[cid, pl.ds(sid * RPT, RPT)])


def _msg_body(g_hbm, e3_hbm, s_hbm,
              row_v, col_v, buf0, buf1, buf2, buf3, zero_v, s_sh,
              g0, g1, g2, g3, s0, s1, s2, s3):
  cid = lax.axis_index("c")
  sid = lax.axis_index("s")
  wid = cid * NS + sid
  nk = jnp.where(wid == NW - 1, KLAST, KMAX)
  bufs = (buf0, buf1, buf2, buf3)
  gsem = (g0, g1, g2, g3)
  ssem = (s0, s1, s2, s3)

  z16 = jnp.zeros((L,), jnp.float32)
  _fill_rows(zero_v, CHUNK, z16)
  for j in range(RPT // CHUNK):
    pltpu.sync_copy(zero_v, s_sh.at[pl.ds(sid * RPT + j * CHUNK, CHUNK)])
  plsc.subcore_barrier()

  _load_edge_chunks(e3_hbm, 0, wid, row_v)
  _load_edge_chunks(e3_hbm, 1, wid, col_v)

  for j in range(3):
    pltpu.async_copy(g_hbm.at[row_v.at[j]], bufs[j], gsem[j])

  def quad(t, _):
    for j in range(4):
      k = 4 * t + j
      pltpu.make_async_copy(g_hbm.at[row_v.at[k]], bufs[j], gsem[j]).wait()
      pltpu.async_copy(bufs[j], s_sh.at[col_v.at[k]], ssem[j], add=True)
      nj = (j + 3) % 4

      @pl.when((k + 3 < nk) & (k > 0))
      def _():
        pltpu.make_async_copy(bufs[nj], s_sh.at[col_v.at[0]], ssem[nj]).wait()
        pltpu.async_copy(g_hbm.at[row_v.at[k + 3]], bufs[nj], gsem[nj])

      if j == 0:
        @pl.when(k == 0)
        def _():
          pltpu.async_copy(g_hbm.at[row_v.at[3]], bufs[3], gsem[3])
    return 0

  lax.fori_loop(0, nk // 4, quad, 0)
  for j in range(4):
    pltpu.make_async_copy(bufs[j], s_sh.at[col_v.at[0]], ssem[j]).wait()
  plsc.subcore_barrier()

  pltpu.sync_copy(s_sh.at[pl.ds(sid * RPT, RPT)],
                  s_hbm.at[pl.ds(sid * RPT, RPT), pl.ds(cid * DH, DH)])


def _tca0_body(x_ref, w_ref, h_ref):
  h_ref[...] = jnp.dot(x_ref[...], w_ref[...],
                       preferred_element_type=jnp.float32)


def _dinv_col(deg_ref):
  d = deg_ref[0] + deg_ref[1]
  rep = jnp.broadcast_to(d[:, None, :], (RBS, 128, 128)).reshape(RB, 128)
  lane = lax.broadcasted_iota(jnp.int32, (RB, 128), 1)
  row = lax.broadcasted_iota(jnp.int32, (RB, 128), 0)
  sel = jnp.where(lane == row % 128, rep, 0.0)
  return lax.rsqrt(jnp.sum(sel, axis=1, keepdims=True) + 1.0)


def _tca1_body(h_ref, deg_ref, g_ref):
  g_ref[...] = h_ref[...] * _dinv_col(deg_ref)


def _tcb_body(s_ref, g_ref, deg_ref, batch_ref, bconv_ref, wlin_ref,
              blin_ref, out_ref, sums, cnts):
  i = pl.program_id(0)

  @pl.when(i == 0)
  def _():
    sums[...] = jnp.zeros_like(sums)
    cnts[...] = jnp.zeros_like(cnts)

  s = s_ref[:, 0:DH] + s_ref[:, DH:2 * DH]
  o = _dinv_col(deg_ref) * (s + g_ref[...]) + bconv_ref[...]
  o = jnp.where(o >= 0, o, 0.01 * o)
  row_ok = (lax.broadcasted_iota(jnp.int32, (RB, 1), 0) + i * RB) < N
  o = jnp.where(row_ok, o, 0.0)

  b = batch_ref[0]
  gid = lax.broadcasted_iota(jnp.int32, (G, RB), 0)
  pt = jnp.where(gid == b, 1.0, 0.0).astype(jnp.float32)
  sums[...] += jnp.dot(pt, o, preferred_element_type=jnp.float32)
  cnts[...] += jnp.sum(pt, axis=1, keepdims=True)

  @pl.when(i == NB - 1)
  def _():
    pooled = sums[...] / jnp.maximum(cnts[...], 1.0)
    emb = jnp.dot(pooled, wlin_ref[...], preferred_element_type=jnp.float32)
    out_ref[...] = jnp.tanh(emb + blin_ref[...])


_tca0_call = pl.pallas_call(
    _tca0_body,
    grid=(NB0,),
    in_specs=[
        pl.BlockSpec((RB0, DIN), lambda i: (i, 0)),
        pl.BlockSpec((DIN, DH), lambda i: (0, 0)),
    ],
    out_specs=pl.BlockSpec((RB0, DH), lambda i: (i, 0)),
    out_shape=jax.ShapeDtypeStruct((NPAD, DH), jnp.float32),
)

_tca1_call = pl.pallas_call(
    _tca1_body,
    grid=(NB,),
    in_specs=[
        pl.BlockSpec((RB, DH), lambda i: (i, 0)),
        pl.BlockSpec((NC, RBS, 128), lambda i: (0, i, 0)),
    ],
    out_specs=pl.BlockSpec((RB, DH), lambda i: (i, 0)),
    out_shape=jax.ShapeDtypeStruct((NPAD, DH), jnp.float32),
)

_tcb_call = pl.pallas_call(
    _tcb_body,
    grid=(NB,),
    in_specs=[
        pl.BlockSpec((RB, NC * DH), lambda i: (i, 0)),
        pl.BlockSpec((RB, DH), lambda i: (i, 0)),
        pl.BlockSpec((NC, RBS, 128), lambda i: (0, i, 0)),
        pl.BlockSpec((1, 1, RB), lambda i: (i, 0, 0)),
        pl.BlockSpec((1, DH), lambda i: (0, 0)),
        pl.BlockSpec((DH, DOUT), lambda i: (0, 0)),
        pl.BlockSpec((1, DOUT), lambda i: (0, 0)),
    ],
    out_specs=pl.BlockSpec((G, DOUT), lambda i: (0, 0)),
    out_shape=jax.ShapeDtypeStruct((G, DOUT), jnp.float32),
    scratch_shapes=[
        pltpu.VMEM((G, G), jnp.float32),
        pltpu.VMEM((G, 1), jnp.float32),
    ],
)


@functools.lru_cache(maxsize=1)
def _sc_kernels():
  mesh = plsc.VectorSubcoreMesh(
      core_axis_name="c", subcore_axis_name="s", num_cores=NC, num_subcores=NS)
  params = pltpu.CompilerParams(use_tc_tiling_on_sc=False)
  deg_kernel = pl.kernel(
      _deg_body,
      out_type=jax.ShapeDtypeStruct((NC, NPAD), jnp.float32),
      mesh=mesh,
      compiler_params=params,
      scratch_types=[
          pltpu.VMEM((KMAX, CHUNK), jnp.int32),
          pltpu.VMEM((CHUNK,), jnp.float32),
          pltpu.VMEM((RPT,), jnp.float32),
          pltpu.VMEM_SHARED((NPAD,), jnp.float32),
      ],
  )
  msg_kernel = pl.kernel(
      _msg_body,
      out_type=jax.ShapeDtypeStruct((NPAD, NC * DH), jnp.float32),
      mesh=mesh,
      compiler_params=params,
      scratch_types=[
          pltpu.VMEM((KMAX, CHUNK), jnp.int32),
          pltpu.VMEM((KMAX, CHUNK), jnp.int32),
          pltpu.VMEM((CHUNK, DH), jnp.float32),
          pltpu.VMEM((CHUNK, DH), jnp.float32),
          pltpu.VMEM((CHUNK, DH), jnp.float32),
          pltpu.VMEM((CHUNK, DH), jnp.float32),
          pltpu.VMEM((CHUNK, DH), jnp.float32),
          pltpu.VMEM_SHARED((NPAD, DH), jnp.float32),
      ] + [pltpu.SemaphoreType.DMA] * 8,
  )
  return deg_kernel, msg_kernel


def kernel(x, edge_index, batch, W_conv, b_conv, W_lin, b_lin):
  deg_kernel, msg_kernel = _sc_kernels()
  e3 = edge_index.reshape(2, NCHUNKS, CHUNK)

  h = _tca0_call(x, W_conv)
  deg = deg_kernel(e3).reshape(NC, NPAD // 128, 128)
  g = _tca1_call(h, deg)
  s = msg_kernel(g, e3)
  batch_p = jnp.full((NPAD,), G, jnp.int32).at[:N].set(batch)
  emb = _tcb_call(s, g, deg, batch_p.reshape(NB, 1, RB),
                  b_conv.reshape(1, DH), W_lin, b_lin.reshape(1, DOUT))
  return emb


# speedup vs baseline: 73.5853x; 1.1582x over previous
import functools

import jax
import jax.numpy as jnp
from jax import lax
from jax.experimental import pallas as pl
from jax.experimental.pallas import tpu as pltpu
from jax.experimental.pallas import tpu_sc as plsc

N = 10000
E = 320000
DIN = 128
DH = 64
DOUT = 32
G = 64

NC = 2
NS = 16
L = 16
NW = NC * NS

CHUNK = 128
NCHUNKS = E // CHUNK
KMAX = 80
KLAST = NCHUNKS - (NW - 1) * KMAX
NPAD = 10240
RPT = NPAD // NS

RB0 = 2000
NB0 = N // RB0
RB = 2048
NB = NPAD // RB
RBS = RB // 128
NSLOT = 5

def _fill_rows(ref, nrows, vec):
  ncols = ref.shape[1] // L

  def body(i, _):
    for j in range(ncols):
      ref[i, pl.ds(j * L, L)] = vec
    return 0

  lax.fori_loop(0, nrows, body, 0)


def _load_edge_chunks(e3_hbm, which, wid, dst_v):
  @pl.when(wid < NW - 1)
  def _():
    pltpu.sync_copy(e3_hbm.at[which, pl.ds(wid * KMAX, KMAX)], dst_v)

  @pl.when(wid == NW - 1)
  def _():
    pltpu.sync_copy(e3_hbm.at[which, pl.ds((NW - 1) * KMAX, KLAST)],
                    dst_v.at[pl.ds(0, KLAST)])


def _deg_body(e3_hbm, deg_hbm, col_v, ones_v, zero_v, deg_sh):
  cid = lax.axis_index("c")
  sid = lax.axis_index("s")
  wid = cid * NS + sid
  nk = jnp.where(wid == NW - 1, KLAST, KMAX)

  one16 = jnp.full((L,), 1.0, jnp.float32)
  z16 = jnp.zeros((L,), jnp.float32)
  for j in range(CHUNK // L):
    ones_v[pl.ds(j * L, L)] = one16
  for j in range(RPT // L):
    zero_v[pl.ds(j * L, L)] = z16

  pltpu.sync_copy(zero_v, deg_sh.at[pl.ds(sid * RPT, RPT)])
  plsc.subcore_barrier()

  _load_edge_chunks(e3_hbm, 1, wid, col_v)

  def step(k, _):
    pltpu.sync_copy(ones_v, deg_sh.at[col_v.at[k]], add=True)
    return 0

  lax.fori_loop(0, nk, step, 0)
  plsc.subcore_barrier()

  pltpu.sync_copy(deg_sh.at[pl.ds(sid * RPT, RPT)],
                  deg_hbm.at[cid, pl.ds(sid * RPT, RPT)])


def _msg_body(g_hbm, e3_hbm, s_hbm, row_v, col_v, *rest):
  bufs = rest[:NSLOT]
  zero_v = rest[NSLOT]
  s_sh = rest[NSLOT + 1]
  gsem = rest[NSLOT + 2:2 * NSLOT + 2]
  ssem = rest[2 * NSLOT + 2:]
  cid = lax.axis_index("c")
  sid = lax.axis_index("s")
  wid = cid * NS + sid
  nk = jnp.where(wid == NW - 1, KLAST, KMAX)

  z16 = jnp.zeros((L,), jnp.float32)
  _fill_rows(zero_v, CHUNK, z16)
  for j in range(RPT // CHUNK):
    pltpu.sync_copy(zero_v, s_sh.at[pl.ds(sid * RPT + j * CHUNK, CHUNK)])
  plsc.subcore_barrier()

  _load_edge_chunks(e3_hbm, 0, wid, row_v)
  _load_edge_chunks(e3_hbm, 1, wid, col_v)

  for j in range(NSLOT - 1):
    pltpu.async_copy(g_hbm.at[row_v.at[j]], bufs[j], gsem[j])

  def rot(t, _):
    for j in range(NSLOT):
      k = NSLOT * t + j
      pltpu.make_async_copy(g_hbm.at[row_v.at[k]], bufs[j], gsem[j]).wait()
      pltpu.async_copy(bufs[j], s_sh.at[col_v.at[k]], ssem[j], add=True)
      nj = (j + NSLOT - 1) % NSLOT

      @pl.when((k + NSLOT - 1 < nk) & (k > 0))
      def _():
        pltpu.make_async_copy(bufs[nj], s_sh.at[col_v.at[0]], ssem[nj]).wait()
        pltpu.async_copy(g_hbm.at[row_v.at[k + NSLOT - 1]], bufs[nj], gsem[nj])

      if j == 0:
        @pl.when(k == 0)
        def _():
          pltpu.async_copy(g_hbm.at[row_v.at[NSLOT - 1]], bufs[NSLOT - 1],
                           gsem[NSLOT - 1])
    return 0

  lax.fori_loop(0, nk // NSLOT, rot, 0)
  for j in range(NSLOT):
    pltpu.make_async_copy(bufs[j], s_sh.at[col_v.at[0]], ssem[j]).wait()
  plsc.subcore_barrier()

  pltpu.sync_copy(s_sh.at[pl.ds(sid * RPT, RPT)],
                  s_hbm.at[pl.ds(sid * RPT, RPT), pl.ds(cid * DH, DH)])


def _tca0_body(x_ref, w_ref, h_ref):
  h_ref[...] = jnp.dot(x_ref[...], w_ref[...],
                       preferred_element_type=jnp.float32)


def _dinv_col(deg_ref):
  d = deg_ref[0] + deg_ref[1]
  rep = jnp.broadcast_to(d[:, None, :], (RBS, 128, 128)).reshape(RB, 128)
  lane = lax.broadcasted_iota(jnp.int32, (RB, 128), 1)
  row = lax.broadcasted_iota(jnp.int32, (RB, 128), 0)
  sel = jnp.where(lane == row % 128, rep, 0.0)
  return lax.rsqrt(jnp.sum(sel, axis=1, keepdims=True) + 1.0)


def _tca1_body(h_ref, deg_ref, g_ref):
  g_ref[...] = h_ref[...] * _dinv_col(deg_ref)


def _tcb_body(s_ref, g_ref, deg_ref, batch_ref, bconv_ref, wlin_ref,
              blin_ref, out_ref, sums, cnts):
  i = pl.program_id(0)

  @pl.when(i == 0)
  def _():
    sums[...] = jnp.zeros_like(sums)
    cnts[...] = jnp.zeros_like(cnts)

  s = s_ref[:, 0:DH] + s_ref[:, DH:2 * DH]
  o = _dinv_col(deg_ref) * (s + g_ref[...]) + bconv_ref[...]
  o = jnp.where(o >= 0, o, 0.01 * o)
  row_ok = (lax.broadcasted_iota(jnp.int32, (RB, 1), 0) + i * RB) < N
  o = jnp.where(row_ok, o, 0.0)

  b = batch_ref[0]
  gid = lax.broadcasted_iota(jnp.int32, (G, RB), 0)
  pt = jnp.where(gid == b, 1.0, 0.0).astype(jnp.float32)
  sums[...] += jnp.dot(pt, o, preferred_element_type=jnp.float32)
  cnts[...] += jnp.sum(pt, axis=1, keepdims=True)

  @pl.when(i == NB - 1)
  def _():
    pooled = sums[...] / jnp.maximum(cnts[...], 1.0)
    emb = jnp.dot(pooled, wlin_ref[...], preferred_element_type=jnp.float32)
    out_ref[...] = jnp.tanh(emb + blin_ref[...])


_tca0_call = pl.pallas_call(
    _tca0_body,
    grid=(NB0,),
    in_specs=[
        pl.BlockSpec((RB0, DIN), lambda i: (i, 0)),
        pl.BlockSpec((DIN, DH), lambda i: (0, 0)),
    ],
    out_specs=pl.BlockSpec((RB0, DH), lambda i: (i, 0)),
    out_shape=jax.ShapeDtypeStruct((NPAD, DH), jnp.float32),
)

_tca1_call = pl.pallas_call(
    _tca1_body,
    grid=(NB,),
    in_specs=[
        pl.BlockSpec((RB, DH), lambda i: (i, 0)),
        pl.BlockSpec((NC, RBS, 128), lambda i: (0, i, 0)),
    ],
    out_specs=pl.BlockSpec((RB, DH), lambda i: (i, 0)),
    out_shape=jax.ShapeDtypeStruct((NPAD, DH), jnp.float32),
)

_tcb_call = pl.pallas_call(
    _tcb_body,
    grid=(NB,),
    in_specs=[
        pl.BlockSpec((RB, NC * DH), lambda i: (i, 0)),
        pl.BlockSpec((RB, DH), lambda i: (i, 0)),
        pl.BlockSpec((NC, RBS, 128), lambda i: (0, i, 0)),
        pl.BlockSpec((1, 1, RB), lambda i: (i, 0, 0)),
        pl.BlockSpec((1, DH), lambda i: (0, 0)),
        pl.BlockSpec((DH, DOUT), lambda i: (0, 0)),
        pl.BlockSpec((1, DOUT), lambda i: (0, 0)),
    ],
    out_specs=pl.BlockSpec((G, DOUT), lambda i: (0, 0)),
    out_shape=jax.ShapeDtypeStruct((G, DOUT), jnp.float32),
    scratch_shapes=[
        pltpu.VMEM((G, G), jnp.float32),
        pltpu.VMEM((G, 1), jnp.float32),
    ],
)


@functools.lru_cache(maxsize=1)
def _sc_kernels():
  mesh = plsc.VectorSubcoreMesh(
      core_axis_name="c", subcore_axis_name="s", num_cores=NC, num_subcores=NS)
  params = pltpu.CompilerParams(use_tc_tiling_on_sc=False)
  deg_kernel = pl.kernel(
      _deg_body,
      out_type=jax.ShapeDtypeStruct((NC, NPAD), jnp.float32),
      mesh=mesh,
      compiler_params=params,
      scratch_types=[
          pltpu.VMEM((KMAX, CHUNK), jnp.int32),
          pltpu.VMEM((CHUNK,), jnp.float32),
          pltpu.VMEM((RPT,), jnp.float32),
          pltpu.VMEM_SHARED((NPAD,), jnp.float32),
      ],
  )
  msg_kernel = pl.kernel(
      _msg_body,
      out_type=jax.ShapeDtypeStruct((NPAD, NC * DH), jnp.float32),
      mesh=mesh,
      compiler_params=params,
      scratch_types=(
          [pltpu.VMEM((KMAX, CHUNK), jnp.int32)] * 2
          + [pltpu.VMEM((CHUNK, DH), jnp.float32)] * (NSLOT + 1)
          + [pltpu.VMEM_SHARED((NPAD, DH), jnp.float32)]
          + [pltpu.SemaphoreType.DMA] * (2 * NSLOT)
      ),
  )
  return deg_kernel, msg_kernel


def kernel(x, edge_index, batch, W_conv, b_conv, W_lin, b_lin):
  deg_kernel, msg_kernel = _sc_kernels()
  e3 = edge_index.reshape(2, NCHUNKS, CHUNK)

  h = _tca0_call(x, W_conv)
  deg = deg_kernel(e3).reshape(NC, NPAD // 128, 128)
  g = _tca1_call(h, deg)
  s = msg_kernel(g, e3)
  batch_p = jnp.full((NPAD,), G, jnp.int32).at[:N].set(batch)
  emb = _tcb_call(s, g, deg, batch_p.reshape(NB, 1, RB),
                  b_conv.reshape(1, DH), W_lin, b_lin.reshape(1, DOUT))
  return emb


# speedup vs baseline: 83.2362x; 1.1312x over previous
import functools

import jax
import jax.numpy as jnp
from jax import lax
from jax.experimental import pallas as pl
from jax.experimental.pallas import tpu as pltpu
from jax.experimental.pallas import tpu_sc as plsc

N = 10000
E = 320000
DIN = 128
DH = 64
DOUT = 32
G = 64

NC = 2
NS = 16
L = 16
NW = NC * NS

CHUNK = 128
NCHUNKS = E // CHUNK
KMAX = 80
KLAST = NCHUNKS - (NW - 1) * KMAX
NPAD = 10240
RPT = NPAD // NS

RB0 = 2000
NB0 = N // RB0
RB = 2048
NB = NPAD // RB
RBS = RB // 128
NSLOT = 5

def _fill_rows_b(ref, nrows, vec):
  ncols = ref.shape[1] // (2 * L)

  def body(i, _):
    for j in range(ncols):
      ref[i, pl.ds(j * 2 * L, 2 * L)] = vec
    return 0

  lax.fori_loop(0, nrows, body, 0)


def _load_edge_chunks(e3_hbm, which, wid, dst_v):
  @pl.when(wid < NW - 1)
  def _():
    pltpu.sync_copy(e3_hbm.at[which, pl.ds(wid * KMAX, KMAX)], dst_v)

  @pl.when(wid == NW - 1)
  def _():
    pltpu.sync_copy(e3_hbm.at[which, pl.ds((NW - 1) * KMAX, KLAST)],
                    dst_v.at[pl.ds(0, KLAST)])


def _deg_body(e3_hbm, deg_hbm, col_v, ones_v, zero_v, deg_sh,
              m0, m1, m2, m3):
  cid = lax.axis_index("c")
  sid = lax.axis_index("s")
  wid = cid * NS + sid
  nk = jnp.where(wid == NW - 1, KLAST, KMAX)
  sems = (m0, m1, m2, m3)

  one16 = jnp.full((L,), 1.0, jnp.float32)
  z16 = jnp.zeros((L,), jnp.float32)
  for j in range(CHUNK // L):
    ones_v[pl.ds(j * L, L)] = one16
  for j in range(RPT // L):
    zero_v[pl.ds(j * L, L)] = z16

  pltpu.sync_copy(zero_v, deg_sh.at[pl.ds(sid * RPT, RPT)])
  plsc.subcore_barrier()

  _load_edge_chunks(e3_hbm, 1, wid, col_v)

  def quad(t, _):
    for j in range(4):
      k = 4 * t + j

      @pl.when(k >= 4)
      def _():
        pltpu.make_async_copy(ones_v, deg_sh.at[col_v.at[0]], sems[j]).wait()

      pltpu.async_copy(ones_v, deg_sh.at[col_v.at[k]], sems[j], add=True)
    return 0

  lax.fori_loop(0, nk // 4, quad, 0)
  for j in range(4):
    pltpu.make_async_copy(ones_v, deg_sh.at[col_v.at[0]], sems[j]).wait()
  plsc.subcore_barrier()

  pltpu.sync_copy(deg_sh.at[pl.ds(sid * RPT, RPT)],
                  deg_hbm.at[cid, pl.ds(sid * RPT, RPT)])


def _msg_body(g_hbm, e3_hbm, s_hbm, row_v, col_v, *rest):
  bufs = rest[:NSLOT]
  zero_v = rest[NSLOT]
  s_sh = rest[NSLOT + 1]
  gsem = rest[NSLOT + 2:2 * NSLOT + 2]
  ssem = rest[2 * NSLOT + 2:]
  cid = lax.axis_index("c")
  sid = lax.axis_index("s")
  wid = cid * NS + sid
  nk = jnp.where(wid == NW - 1, KLAST, KMAX)

  z32 = jnp.zeros((2 * L,), jnp.bfloat16)
  _fill_rows_b(zero_v, CHUNK, z32)
  for j in range(RPT // CHUNK):
    pltpu.sync_copy(zero_v, s_sh.at[pl.ds(sid * RPT + j * CHUNK, CHUNK)])
  plsc.subcore_barrier()

  _load_edge_chunks(e3_hbm, 0, wid, row_v)
  _load_edge_chunks(e3_hbm, 1, wid, col_v)

  for j in range(NSLOT - 1):
    pltpu.async_copy(g_hbm.at[row_v.at[j]], bufs[j], gsem[j])

  def rot(t, _):
    for j in range(NSLOT):
      k = NSLOT * t + j
      pltpu.make_async_copy(g_hbm.at[row_v.at[k]], bufs[j], gsem[j]).wait()
      pltpu.async_copy(bufs[j], s_sh.at[col_v.at[k]], ssem[j], add=True)
      nj = (j + NSLOT - 1) % NSLOT

      @pl.when((k + NSLOT - 1 < nk) & (k > 0))
      def _():
        pltpu.make_async_copy(bufs[nj], s_sh.at[col_v.at[0]], ssem[nj]).wait()
        pltpu.async_copy(g_hbm.at[row_v.at[k + NSLOT - 1]], bufs[nj], gsem[nj])

      if j == 0:
        @pl.when(k == 0)
        def _():
          pltpu.async_copy(g_hbm.at[row_v.at[NSLOT - 1]], bufs[NSLOT - 1],
                           gsem[NSLOT - 1])
    return 0

  lax.fori_loop(0, nk // NSLOT, rot, 0)
  for j in range(NSLOT):
    pltpu.make_async_copy(bufs[j], s_sh.at[col_v.at[0]], ssem[j]).wait()
  plsc.subcore_barrier()

  pltpu.sync_copy(s_sh.at[pl.ds(sid * RPT, RPT)],
                  s_hbm.at[pl.ds(sid * RPT, RPT), pl.ds(cid * DH, DH)])


def _tca0_body(x_ref, w_ref, h_ref):
  h_ref[...] = jnp.dot(x_ref[...], w_ref[...],
                       preferred_element_type=jnp.float32)


def _dinv_col(deg_ref):
  d = deg_ref[0] + deg_ref[1]
  rep = jnp.broadcast_to(d[:, None, :], (RBS, 128, 128)).reshape(RB, 128)
  lane = lax.broadcasted_iota(jnp.int32, (RB, 128), 1)
  row = lax.broadcasted_iota(jnp.int32, (RB, 128), 0)
  sel = jnp.where(lane == row % 128, rep, 0.0)
  return lax.rsqrt(jnp.sum(sel, axis=1, keepdims=True) + 1.0)


def _tca1_body(h_ref, deg_ref, g_ref):
  g_ref[...] = (h_ref[...] * _dinv_col(deg_ref)).astype(jnp.bfloat16)


def _tcb_body(s_ref, g_ref, deg_ref, batch_ref, bconv_ref, wlin_ref,
              blin_ref, out_ref, sums, cnts):
  i = pl.program_id(0)

  @pl.when(i == 0)
  def _():
    sums[...] = jnp.zeros_like(sums)
    cnts[...] = jnp.zeros_like(cnts)

  s = (s_ref[:, 0:DH].astype(jnp.float32)
       + s_ref[:, DH:2 * DH].astype(jnp.float32))
  o = _dinv_col(deg_ref) * (s + g_ref[...].astype(jnp.float32)) \
      + bconv_ref[...]
  o = jnp.where(o >= 0, o, 0.01 * o)
  row_ok = (lax.broadcasted_iota(jnp.int32, (RB, 1), 0) + i * RB) < N
  o = jnp.where(row_ok, o, 0.0)

  b = batch_ref[0]
  gid = lax.broadcasted_iota(jnp.int32, (G, RB), 0)
  pt = jnp.where(gid == b, 1.0, 0.0).astype(jnp.float32)
  sums[...] += jnp.dot(pt, o, preferred_element_type=jnp.float32)
  cnts[...] += jnp.sum(pt, axis=1, keepdims=True)

  @pl.when(i == NB - 1)
  def _():
    pooled = sums[...] / jnp.maximum(cnts[...], 1.0)
    emb = jnp.dot(pooled, wlin_ref[...], preferred_element_type=jnp.float32)
    out_ref[...] = jnp.tanh(emb + blin_ref[...])


_tca0_call = pl.pallas_call(
    _tca0_body,
    grid=(NB0,),
    in_specs=[
        pl.BlockSpec((RB0, DIN), lambda i: (i, 0)),
        pl.BlockSpec((DIN, DH), lambda i: (0, 0)),
    ],
    out_specs=pl.BlockSpec((RB0, DH), lambda i: (i, 0)),
    out_shape=jax.ShapeDtypeStruct((NPAD, DH), jnp.float32),
)

_tca1_call = pl.pallas_call(
    _tca1_body,
    grid=(NB,),
    in_specs=[
        pl.BlockSpec((RB, DH), lambda i: (i, 0)),
        pl.BlockSpec((NC, RBS, 128), lambda i: (0, i, 0)),
    ],
    out_specs=pl.BlockSpec((RB, DH), lambda i: (i, 0)),
    out_shape=jax.ShapeDtypeStruct((NPAD, DH), jnp.bfloat16),
)

_tcb_call = pl.pallas_call(
    _tcb_body,
    grid=(NB,),
    in_specs=[
        pl.BlockSpec((RB, NC * DH), lambda i: (i, 0)),
        pl.BlockSpec((RB, DH), lambda i: (i, 0)),
        pl.BlockSpec((NC, RBS, 128), lambda i: (0, i, 0)),
        pl.BlockSpec((1, 1, RB), lambda i: (i, 0, 0)),
        pl.BlockSpec((1, DH), lambda i: (0, 0)),
        pl.BlockSpec((DH, DOUT), lambda i: (0, 0)),
        pl.BlockSpec((1, DOUT), lambda i: (0, 0)),
    ],
    out_specs=pl.BlockSpec((G, DOUT), lambda i: (0, 0)),
    out_shape=jax.ShapeDtypeStruct((G, DOUT), jnp.float32),
    scratch_shapes=[
        pltpu.VMEM((G, G), jnp.float32),
        pltpu.VMEM((G, 1), jnp.float32),
    ],
)


@functools.lru_cache(maxsize=1)
def _sc_kernels():
  mesh = plsc.VectorSubcoreMesh(
      core_axis_name="c", subcore_axis_name="s", num_cores=NC, num_subcores=NS)
  params = pltpu.CompilerParams(use_tc_tiling_on_sc=False)
  deg_kernel = pl.kernel(
      _deg_body,
      out_type=jax.ShapeDtypeStruct((NC, NPAD), jnp.float32),
      mesh=mesh,
      compiler_params=params,
      scratch_types=[
          pltpu.VMEM((KMAX, CHUNK), jnp.int32),
          pltpu.VMEM((CHUNK,), jnp.float32),
          pltpu.VMEM((RPT,), jnp.float32),
          pltpu.VMEM_SHARED((NPAD,), jnp.float32),
      ] + [pltpu.SemaphoreType.DMA] * 4,
  )
  msg_kernel = pl.kernel(
      _msg_body,
      out_type=jax.ShapeDtypeStruct((NPAD, NC * DH), jnp.bfloat16),
      mesh=mesh,
      compiler_params=params,
      scratch_types=(
          [pltpu.VMEM((KMAX, CHUNK), jnp.int32)] * 2
          + [pltpu.VMEM((CHUNK, DH), jnp.bfloat16)] * (NSLOT + 1)
          + [pltpu.VMEM_SHARED((NPAD, DH), jnp.bfloat16)]
          + [pltpu.SemaphoreType.DMA] * (2 * NSLOT)
      ),
  )
  return deg_kernel, msg_kernel


def kernel(x, edge_index, batch, W_conv, b_conv, W_lin, b_lin):
  deg_kernel, msg_kernel = _sc_kernels()
  e3 = edge_index.reshape(2, NCHUNKS, CHUNK)

  h = _tca0_call(x, W_conv)
  deg = deg_kernel(e3).reshape(NC, NPAD // 128, 128)
  g = _tca1_call(h, deg)
  s = msg_kernel(g, e3)
  batch_p = jnp.full((NPAD,), G, jnp.int32).at[:N].set(batch)
  emb = _tcb_call(s, g, deg, batch_p.reshape(NB, 1, RB),
                  b_conv.reshape(1, DH), W_lin, b_lin.reshape(1, DOUT))
  return emb


# speedup vs baseline: 85.0717x; 1.0221x over previous
import functools

import jax
import jax.numpy as jnp
from jax import lax
from jax.experimental import pallas as pl
from jax.experimental.pallas import tpu as pltpu
from jax.experimental.pallas import tpu_sc as plsc

N = 10000
E = 320000
DIN = 128
DH = 64
DOUT = 32
G = 64

NC = 2
NS = 16
L = 16
NW = NC * NS

CHUNK = 128
NCHUNKS = E // CHUNK
KMAX = 80
KLAST = NCHUNKS - (NW - 1) * KMAX
NPAD = 10240
RPT = NPAD // NS

RB0 = 2000
NB0 = N // RB0
RB = 2048
NB = NPAD // RB
RBS = RB // 128
NSLOT = 10

def _fill_rows_b(ref, nrows, vec):
  ncols = ref.shape[1] // (2 * L)

  def body(i, _):
    for j in range(ncols):
      ref[i, pl.ds(j * 2 * L, 2 * L)] = vec
    return 0

  lax.fori_loop(0, nrows, body, 0)


def _load_edge_chunks(e3_hbm, which, wid, dst_v):
  @pl.when(wid < NW - 1)
  def _():
    pltpu.sync_copy(e3_hbm.at[which, pl.ds(wid * KMAX, KMAX)], dst_v)

  @pl.when(wid == NW - 1)
  def _():
    pltpu.sync_copy(e3_hbm.at[which, pl.ds((NW - 1) * KMAX, KLAST)],
                    dst_v.at[pl.ds(0, KLAST)])


def _deg_body(e3_hbm, deg_hbm, col_v, ones_v, zero_v, deg_sh,
              m0, m1, m2, m3):
  cid = lax.axis_index("c")
  sid = lax.axis_index("s")
  wid = cid * NS + sid
  nk = jnp.where(wid == NW - 1, KLAST, KMAX)
  sems = (m0, m1, m2, m3)

  one16 = jnp.full((L,), 1.0, jnp.float32)
  z16 = jnp.zeros((L,), jnp.float32)
  for j in range(CHUNK // L):
    ones_v[pl.ds(j * L, L)] = one16
  for j in range(RPT // L):
    zero_v[pl.ds(j * L, L)] = z16

  pltpu.sync_copy(zero_v, deg_sh.at[pl.ds(sid * RPT, RPT)])
  plsc.subcore_barrier()

  _load_edge_chunks(e3_hbm, 1, wid, col_v)

  def quad(t, _):
    for j in range(4):
      k = 4 * t + j

      @pl.when(k >= 4)
      def _():
        pltpu.make_async_copy(ones_v, deg_sh.at[col_v.at[0]], sems[j]).wait()

      pltpu.async_copy(ones_v, deg_sh.at[col_v.at[k]], sems[j], add=True)
    return 0

  lax.fori_loop(0, nk // 4, quad, 0)
  for j in range(4):
    pltpu.make_async_copy(ones_v, deg_sh.at[col_v.at[0]], sems[j]).wait()
  plsc.subcore_barrier()

  pltpu.sync_copy(deg_sh.at[pl.ds(sid * RPT, RPT)],
                  deg_hbm.at[cid, pl.ds(sid * RPT, RPT)])


def _msg_body(g_hbm, e3_hbm, s_hbm, row_v, col_v, *rest):
  bufs = rest[:NSLOT]
  zero_v = rest[NSLOT]
  s_sh = rest[NSLOT + 1]
  gsem = rest[NSLOT + 2:2 * NSLOT + 2]
  ssem = rest[2 * NSLOT + 2:]
  cid = lax.axis_index("c")
  sid = lax.axis_index("s")
  wid = cid * NS + sid
  nk = jnp.where(wid == NW - 1, KLAST, KMAX)

  z32 = jnp.zeros((2 * L,), jnp.bfloat16)
  _fill_rows_b(zero_v, CHUNK, z32)
  for j in range(RPT // CHUNK):
    pltpu.sync_copy(zero_v, s_sh.at[pl.ds(sid * RPT + j * CHUNK, CHUNK)])
  plsc.subcore_barrier()

  _load_edge_chunks(e3_hbm, 0, wid, row_v)
  _load_edge_chunks(e3_hbm, 1, wid, col_v)

  for j in range(NSLOT - 1):
    pltpu.async_copy(g_hbm.at[row_v.at[j]], bufs[j], gsem[j])

  def rot(t, _):
    for j in range(NSLOT):
      k = NSLOT * t + j
      pltpu.make_async_copy(g_hbm.at[row_v.at[k]], bufs[j], gsem[j]).wait()
      pltpu.async_copy(bufs[j], s_sh.at[col_v.at[k]], ssem[j], add=True)
      nj = (j + NSLOT - 1) % NSLOT

      @pl.when((k + NSLOT - 1 < nk) & (k > 0))
      def _():
        pltpu.make_async_copy(bufs[nj], s_sh.at[col_v.at[0]], ssem[nj]).wait()
        pltpu.async_copy(g_hbm.at[row_v.at[k + NSLOT - 1]], bufs[nj], gsem[nj])

      if j == 0:
        @pl.when(k == 0)
        def _():
          pltpu.async_copy(g_hbm.at[row_v.at[NSLOT - 1]], bufs[NSLOT - 1],
                           gsem[NSLOT - 1])
    return 0

  lax.fori_loop(0, nk // NSLOT, rot, 0)
  for j in range(NSLOT):
    pltpu.make_async_copy(bufs[j], s_sh.at[col_v.at[0]], ssem[j]).wait()
  plsc.subcore_barrier()

  pltpu.sync_copy(s_sh.at[pl.ds(sid * RPT, RPT)],
                  s_hbm.at[pl.ds(sid * RPT, RPT), pl.ds(cid * DH, DH)])


def _tca0_body(x_ref, w_ref, h_ref):
  h_ref[...] = jnp.dot(x_ref[...], w_ref[...],
                       preferred_element_type=jnp.float32)


def _dinv_col(deg_ref):
  d = deg_ref[0] + deg_ref[1]
  rep = jnp.broadcast_to(d[:, None, :], (RBS, 128, 128)).reshape(RB, 128)
  lane = lax.broadcasted_iota(jnp.int32, (RB, 128), 1)
  row = lax.broadcasted_iota(jnp.int32, (RB, 128), 0)
  sel = jnp.where(lane == row % 128, rep, 0.0)
  return lax.rsqrt(jnp.sum(sel, axis=1, keepdims=True) + 1.0)


def _tca1_body(h_ref, deg_ref, g_ref):
  g_ref[...] = (h_ref[...] * _dinv_col(deg_ref)).astype(jnp.bfloat16)


def _tcb_body(s_ref, g_ref, deg_ref, batch_ref, bconv_ref, wlin_ref,
              blin_ref, out_ref, sums, cnts):
  i = pl.program_id(0)

  @pl.when(i == 0)
  def _():
    sums[...] = jnp.zeros_like(sums)
    cnts[...] = jnp.zeros_like(cnts)

  s = (s_ref[:, 0:DH].astype(jnp.float32)
       + s_ref[:, DH:2 * DH].astype(jnp.float32))
  o = _dinv_col(deg_ref) * (s + g_ref[...].astype(jnp.float32)) \
      + bconv_ref[...]
  o = jnp.where(o >= 0, o, 0.01 * o)
  row_ok = (lax.broadcasted_iota(jnp.int32, (RB, 1), 0) + i * RB) < N
  o = jnp.where(row_ok, o, 0.0)

  b = batch_ref[0]
  gid = lax.broadcasted_iota(jnp.int32, (G, RB), 0)
  pt = jnp.where(gid == b, 1.0, 0.0).astype(jnp.float32)
  sums[...] += jnp.dot(pt, o, preferred_element_type=jnp.float32)
  cnts[...] += jnp.sum(pt, axis=1, keepdims=True)

  @pl.when(i == NB - 1)
  def _():
    pooled = sums[...] / jnp.maximum(cnts[...], 1.0)
    emb = jnp.dot(pooled, wlin_ref[...], preferred_element_type=jnp.float32)
    out_ref[...] = jnp.tanh(emb + blin_ref[...])


_tca0_call = pl.pallas_call(
    _tca0_body,
    grid=(NB0,),
    in_specs=[
        pl.BlockSpec((RB0, DIN), lambda i: (i, 0)),
        pl.BlockSpec((DIN, DH), lambda i: (0, 0)),
    ],
    out_specs=pl.BlockSpec((RB0, DH), lambda i: (i, 0)),
    out_shape=jax.ShapeDtypeStruct((NPAD, DH), jnp.float32),
)

_tca1_call = pl.pallas_call(
    _tca1_body,
    grid=(NB,),
    in_specs=[
        pl.BlockSpec((RB, DH), lambda i: (i, 0)),
        pl.BlockSpec((NC, RBS, 128), lambda i: (0, i, 0)),
    ],
    out_specs=pl.BlockSpec((RB, DH), lambda i: (i, 0)),
    out_shape=jax.ShapeDtypeStruct((NPAD, DH), jnp.bfloat16),
)

_tcb_call = pl.pallas_call(
    _tcb_body,
    grid=(NB,),
    in_specs=[
        pl.BlockSpec((RB, NC * DH), lambda i: (i, 0)),
        pl.BlockSpec((RB, DH), lambda i: (i, 0)),
        pl.BlockSpec((NC, RBS, 128), lambda i: (0, i, 0)),
        pl.BlockSpec((1, 1, RB), lambda i: (i, 0, 0)),
        pl.BlockSpec((1, DH), lambda i: (0, 0)),
        pl.BlockSpec((DH, DOUT), lambda i: (0, 0)),
        pl.BlockSpec((1, DOUT), lambda i: (0, 0)),
    ],
    out_specs=pl.BlockSpec((G, DOUT), lambda i: (0, 0)),
    out_shape=jax.ShapeDtypeStruct((G, DOUT), jnp.float32),
    scratch_shapes=[
        pltpu.VMEM((G, G), jnp.float32),
        pltpu.VMEM((G, 1), jnp.float32),
    ],
)


@functools.lru_cache(maxsize=1)
def _sc_kernels():
  mesh = plsc.VectorSubcoreMesh(
      core_axis_name="c", subcore_axis_name="s", num_cores=NC, num_subcores=NS)
  params = pltpu.CompilerParams(use_tc_tiling_on_sc=False)
  deg_kernel = pl.kernel(
      _deg_body,
      out_type=jax.ShapeDtypeStruct((NC, NPAD), jnp.float32),
      mesh=mesh,
      compiler_params=params,
      scratch_types=[
          pltpu.VMEM((KMAX, CHUNK), jnp.int32),
          pltpu.VMEM((CHUNK,), jnp.float32),
          pltpu.VMEM((RPT,), jnp.float32),
          pltpu.VMEM_SHARED((NPAD,), jnp.float32),
      ] + [pltpu.SemaphoreType.DMA] * 4,
  )
  msg_kernel = pl.kernel(
      _msg_body,
      out_type=jax.ShapeDtypeStruct((NPAD, NC * DH), jnp.bfloat16),
      mesh=mesh,
      compiler_params=params,
      scratch_types=(
          [pltpu.VMEM((KMAX, CHUNK), jnp.int32)] * 2
          + [pltpu.VMEM((CHUNK, DH), jnp.bfloat16)] * (NSLOT + 1)
          + [pltpu.VMEM_SHARED((NPAD, DH), jnp.bfloat16)]
          + [pltpu.SemaphoreType.DMA] * (2 * NSLOT)
      ),
  )
  return deg_kernel, msg_kernel


def kernel(x, edge_index, batch, W_conv, b_conv, W_lin, b_lin):
  deg_kernel, msg_kernel = _sc_kernels()
  e3 = edge_index.reshape(2, NCHUNKS, CHUNK)

  h = _tca0_call(x, W_conv)
  deg = deg_kernel(e3).reshape(NC, NPAD // 128, 128)
  g = _tca1_call(h, deg)
  s = msg_kernel(g, e3)
  batch_p = jnp.full((NPAD,), G, jnp.int32).at[:N].set(batch)
  emb = _tcb_call(s, g, deg, batch_p.reshape(NB, 1, RB),
                  b_conv.reshape(1, DH), W_lin, b_lin.reshape(1, DOUT))
  return emb
